# baseline (speedup 1.0000x reference)
import functools

import jax
import jax.numpy as jnp
from jax import lax
from jax.experimental import pallas as pl
from jax.experimental.pallas import tpu as pltpu
from jax.experimental.pallas import tpu_sc as plsc

_INF = 100000000.0
_LEVELS = (16384, 4096, 1024, 256, 64)
_STARTS = (0, 16384, 20480, 21504, 21760)
_N = 21824
_G = 100
_GPAD = 112
_K = 9
_NREC = 80
_PER_TILE = 1376
_NPC = 16 * _PER_TILE
_GPC = 50
_BIG = 3e38


def _lane16():
    return lax.broadcasted_iota(jnp.int32, (16,), 0)


def _splat_from(ref, j):
    return plsc.load_gather(ref, [jnp.full((16,), j, jnp.int32)])


def _splat_lane(vec, j, tmp_ref):
    tmp_ref[...] = vec
    return _splat_from(tmp_ref, j)


def _hsum(vec, tmp_ref):
    lane = _lane16()
    for m in (8, 4, 2, 1):
        tmp_ref[...] = vec
        vec = vec + plsc.load_gather(tmp_ref, [lane ^ m])
    return vec


def _sqrt_newton(x):
    i = lax.bitcast_convert_type(x, jnp.int32)
    y = lax.bitcast_convert_type((i >> 1) + jnp.int32(0x1FBD1DF6), jnp.float32)
    for _ in range(4):
        y = 0.5 * (y + x / y)
    return jnp.where(x > 0.0, y, 0.0)


def _merge_topk(tk, tv, d2, idx):
    dk, dv = plsc.sort_key_val(d2, idx)
    dk = lax.rev(dk, (0,))
    dv = lax.rev(dv, (0,))
    take_t = tk <= dk
    lk = jnp.where(take_t, tk, dk)
    lv = jnp.where(take_t, tv, dv)
    return plsc.sort_key_val(lk, lv)


def _scan_level(cx_v, cy_v, gcx, gcy, start, n_batches, tmp_v):
    lane = _lane16()

    def body(b, carry):
        tk, tv, t9v = carry
        off = start + b * 32
        dxa = cx_v[pl.ds(off, 16)] - gcx
        dya = cy_v[pl.ds(off, 16)] - gcy
        d2a = dxa * dxa + dya * dya
        dxb = cx_v[pl.ds(off + 16, 16)] - gcx
        dyb = cy_v[pl.ds(off + 16, 16)] - gcy
        d2b = dxb * dxb + dyb * dyb
        below = (d2a < t9v) | (d2b < t9v)
        cnt = plsc.all_reduce_population_count(below)

        def do_merge(_):
            tk2, tv2 = _merge_topk(tk, tv, d2a, off + lane)
            tk3, tv3 = _merge_topk(tk2, tv2, d2b, off + 16 + lane)
            return tk3, tv3, _splat_lane(tk3, 8, tmp_v)

        def skip(_):
            return tk, tv, t9v

        return lax.cond(cnt[0] > 0, do_merge, skip, 0)

    tk0 = jnp.full((16,), _BIG, jnp.float32)
    tv0 = jnp.zeros((16,), jnp.int32)
    tk, tv, _ = lax.fori_loop(0, n_batches // 2, body,
                              (tk0, tv0, jnp.full((16,), _BIG, jnp.float32)))
    return tk, tv


def _sc_body(x0_h, y0_h, x1_h, y1_h, g0_h, g1_h, g2_h, g3_h,
             pmax_h, parg_h, recp_h, reci_h,
             cx_v, cy_v, st_v, g0_v, g1_v, g2_v, g3_v,
             ids_v, pov_v, bx0_v, by0_v, bx1_v, by1_v,
             recp_v, reci_v, best_v, bestg_v, tmp_v, sem):
    c = lax.axis_index("c")
    s = lax.axis_index("s")
    lane = _lane16()
    valid9 = lane < _K

    pltpu.sync_copy(x0_h, cx_v)
    pltpu.sync_copy(x1_h, st_v)

    def _cb(i, _):
        o = i * 16
        cx_v[pl.ds(o, 16)] = (cx_v[pl.ds(o, 16)] + st_v[pl.ds(o, 16)]) * 0.5
        return 0

    lax.fori_loop(0, _N // 16, _cb, 0)
    pltpu.sync_copy(y0_h, cy_v)
    pltpu.sync_copy(y1_h, st_v)

    def _cb2(i, _):
        o = i * 16
        cy_v[pl.ds(o, 16)] = (cy_v[pl.ds(o, 16)] + st_v[pl.ds(o, 16)]) * 0.5
        return 0

    lax.fori_loop(0, _N // 16, _cb2, 0)

    pltpu.sync_copy(g0_h, g0_v)
    pltpu.sync_copy(g1_h, g1_v)
    pltpu.sync_copy(g2_h, g2_v)
    pltpu.sync_copy(g3_h, g3_v)

    def _per_gt(k, _):
        g = c * _GPC + s + 16 * k
        gx0 = _splat_from(g0_v, g)
        gy0 = _splat_from(g1_v, g)
        gx1 = _splat_from(g2_v, g)
        gy1 = _splat_from(g3_v, g)
        gcx = (gx0 + gx1) * 0.5
        gcy = (gy0 + gy1) * 0.5

        for lvl in range(5):
            tk, tv = _scan_level(cx_v, cy_v, gcx, gcy, _STARTS[lvl],
                                 _LEVELS[lvl] // 16, tmp_v)
            ids_v[pl.ds(lvl * 16, 16)] = tv

        cps = [pltpu.async_copy(x0_h.at[ids_v], bx0_v, sem),
               pltpu.async_copy(y0_h.at[ids_v], by0_v, sem),
               pltpu.async_copy(x1_h.at[ids_v], bx1_v, sem),
               pltpu.async_copy(y1_h.at[ids_v], by1_v, sem)]
        for cp in cps:
            cp.wait()

        ovs = []
        ssum = jnp.zeros((16,), jnp.float32)
        for lvl in range(5):
            ax0 = bx0_v[pl.ds(lvl * 16, 16)]
            ay0 = by0_v[pl.ds(lvl * 16, 16)]
            ax1 = bx1_v[pl.ds(lvl * 16, 16)]
            ay1 = by1_v[pl.ds(lvl * 16, 16)]
            iw = jnp.maximum(jnp.minimum(ax1, gx1) - jnp.maximum(ax0, gx0),
                             0.0)
            ih = jnp.maximum(jnp.minimum(ay1, gy1) - jnp.maximum(ay0, gy0),
                             0.0)
            inter = iw * ih
            area_a = (ax1 - ax0) * (ay1 - ay0)
            area_g = (gx1 - gx0) * (gy1 - gy0)
            ov = inter / jnp.maximum(area_a + area_g - inter, 1e-6)
            acx = (ax0 + ax1) * 0.5
            acy = (ay0 + ay1) * 0.5
            dmin = jnp.minimum(jnp.minimum(acx - gx0, acy - gy0),
                               jnp.minimum(gx1 - acx, gy1 - acy))
            ovs.append((ov, dmin))
            ssum = ssum + _hsum(jnp.where(valid9, ov, 0.0), tmp_v)

        mean = ssum / 45.0
        vsum = jnp.zeros((16,), jnp.float32)
        for ov, _dm in ovs:
            dev = ov - mean
            vsum = vsum + _hsum(jnp.where(valid9, dev * dev, 0.0), tmp_v)
        thr = mean + _sqrt_newton(vsum / 44.0)

        for lvl in range(5):
            ov, dmin = ovs[lvl]
            pos = valid9 & (ov >= thr) & (dmin > 0.01)
            pov_v[pl.ds(lvl * 16, 16)] = jnp.where(pos, ov, -1.0)

        pltpu.sync_copy(pov_v, recp_h.at[pl.ds(g * _NREC, _NREC)])
        pltpu.sync_copy(ids_v, reci_h.at[pl.ds(g * _NREC, _NREC)])
        return 0

    n_gts = (_GPC - s + 15) // 16
    lax.fori_loop(0, n_gts, _per_gt, 0)

    plsc.subcore_barrier()
    pltpu.sync_copy(recp_h.at[pl.ds(c * _GPC * _NREC, _GPC * _NREC)], recp_v)
    pltpu.sync_copy(reci_h.at[pl.ds(c * _GPC * _NREC, _GPC * _NREC)], reci_v)
    base = s * _PER_TILE

    def _init(i, _):
        o = i * 16
        best_v[pl.ds(o, 16)] = jnp.full((16,), -_INF, jnp.float32)
        bestg_v[pl.ds(o, 16)] = jnp.zeros((16,), jnp.int32)
        return 0

    lax.fori_loop(0, _PER_TILE // 16, _init, 0)

    def _mb(t, _):
        off = t * 16
        pov = recp_v[pl.ds(off, 16)]
        ids = reci_v[pl.ds(off, 16)]
        loc = ids - base
        m = (loc >= 0) & (loc < _PER_TILE) & (pov >= 0.0)
        cur = plsc.load_gather(best_v, [loc], mask=m)
        upd = m & (pov > cur)
        gvec = jnp.full((16,), c * _GPC + t // 5, jnp.int32)
        plsc.store_scatter(best_v, [loc], pov, mask=upd)
        plsc.store_scatter(bestg_v, [loc], gvec, mask=upd)
        return 0

    lax.fori_loop(0, _GPC * 5, _mb, 0)

    pltpu.sync_copy(best_v, pmax_h.at[pl.ds(c * _NPC + base, _PER_TILE)])
    pltpu.sync_copy(bestg_v, parg_h.at[pl.ds(c * _NPC + base, _PER_TILE)])


def _combine_body(pm_ref, pa_ref, mo_ref, ag_ref):
    m0 = pm_ref[0:1, :]
    m1 = pm_ref[1:2, :]
    use0 = m0 >= m1
    mo_ref[...] = jnp.where(use0, m0, m1)
    ag_ref[...] = jnp.where(use0, pa_ref[0:1, :], pa_ref[1:2, :])


def kernel(anchors, gt_bboxes):
    at = anchors.T
    x0, y0, x1, y1 = at[0], at[1], at[2], at[3]
    gpad = jnp.zeros((4, _GPAD), jnp.float32).at[:, :_G].set(gt_bboxes.T)

    mesh = plsc.VectorSubcoreMesh(core_axis_name="c", subcore_axis_name="s",
                                  num_cores=2, num_subcores=16)
    sc = pl.kernel(
        _sc_body,
        out_type=[
            jax.ShapeDtypeStruct((2 * _NPC,), jnp.float32),
            jax.ShapeDtypeStruct((2 * _NPC,), jnp.int32),
            jax.ShapeDtypeStruct((_G * _NREC,), jnp.float32),
            jax.ShapeDtypeStruct((_G * _NREC,), jnp.int32),
        ],
        mesh=mesh,
        compiler_params=pltpu.CompilerParams(needs_layout_passes=False),
        scratch_types=[
            pltpu.VMEM((_N,), jnp.float32),
            pltpu.VMEM((_N,), jnp.float32),
            pltpu.VMEM((_N,), jnp.float32),
            pltpu.VMEM((_GPAD,), jnp.float32),
            pltpu.VMEM((_GPAD,), jnp.float32),
            pltpu.VMEM((_GPAD,), jnp.float32),
            pltpu.VMEM((_GPAD,), jnp.float32),
            pltpu.VMEM((_NREC,), jnp.int32),
            pltpu.VMEM((_NREC,), jnp.float32),
            pltpu.VMEM((_NREC,), jnp.float32),
            pltpu.VMEM((_NREC,), jnp.float32),
            pltpu.VMEM((_NREC,), jnp.float32),
            pltpu.VMEM((_NREC,), jnp.float32),
            pltpu.VMEM((_GPC * _NREC,), jnp.float32),
            pltpu.VMEM((_GPC * _NREC,), jnp.int32),
            pltpu.VMEM((_PER_TILE,), jnp.float32),
            pltpu.VMEM((_PER_TILE,), jnp.int32),
            pltpu.VMEM((16,), jnp.float32),
            pltpu.SemaphoreType.DMA,
        ],
    )
    pm, pa, _rp, _ri = sc(x0, y0, x1, y1,
                          gpad[0], gpad[1], gpad[2], gpad[3])

    mo, ag = pl.pallas_call(
        _combine_body,
        out_shape=(
            jax.ShapeDtypeStruct((1, _NPC), jnp.float32),
            jax.ShapeDtypeStruct((1, _NPC), jnp.int32),
        ),
    )(pm.reshape(2, _NPC), pa.reshape(2, _NPC))
    return mo[0, :_N], ag[0, :_N]


# speedup vs baseline: 1.3269x; 1.3269x over previous
import functools

import jax
import jax.numpy as jnp
from jax import lax
from jax.experimental import pallas as pl
from jax.experimental.pallas import tpu as pltpu
from jax.experimental.pallas import tpu_sc as plsc

_INF = 100000000.0
_LEVELS = (16384, 4096, 1024, 256, 64)
_STARTS = (0, 16384, 20480, 21504, 21760)
_N = 21824
_G = 100
_GPAD = 112
_K = 9
_NREC = 80
_PER_TILE = 1376
_NPC = 16 * _PER_TILE
_GPC = 50
_BIG = 3e38


def _lane16():
    return lax.broadcasted_iota(jnp.int32, (16,), 0)


def _splat_from(ref, j):
    return plsc.load_gather(ref, [jnp.full((16,), j, jnp.int32)])


def _splat_lane(vec, j, tmp_ref):
    tmp_ref[...] = vec
    return _splat_from(tmp_ref, j)


def _hsum(vec, tmp_ref):
    lane = _lane16()
    for m in (8, 4, 2, 1):
        tmp_ref[...] = vec
        vec = vec + plsc.load_gather(tmp_ref, [lane ^ m])
    return vec


def _sqrt_newton(x):
    i = lax.bitcast_convert_type(x, jnp.int32)
    y = lax.bitcast_convert_type((i >> 1) + jnp.int32(0x1FBD1DF6), jnp.float32)
    for _ in range(4):
        y = 0.5 * (y + x / y)
    return jnp.where(x > 0.0, y, 0.0)


def _merge_topk(tk, tv, d2, idx):
    dk, dv = plsc.sort_key_val(d2, idx)
    dk = lax.rev(dk, (0,))
    dv = lax.rev(dv, (0,))
    take_t = tk <= dk
    lk = jnp.where(take_t, tk, dk)
    lv = jnp.where(take_t, tv, dv)
    return plsc.sort_key_val(lk, lv)


def _scan_level(cx_v, cy_v, gcx, gcy, start, n_batches, tmp_v):
    lane = _lane16()

    def body(b, carry):
        tk, tv, t9v = carry
        off = start + b * 64
        d2s = []
        below = None
        for u in range(4):
            dx = cx_v[pl.ds(off + u * 16, 16)] - gcx
            dy = cy_v[pl.ds(off + u * 16, 16)] - gcy
            d2 = dx * dx + dy * dy
            d2s.append(d2)
            bl = d2 < t9v
            below = bl if below is None else (below | bl)
        cnt = plsc.all_reduce_population_count(below)

        def do_merge(_):
            tkx, tvx = tk, tv
            for u in range(4):
                tkx, tvx = _merge_topk(tkx, tvx, d2s[u], off + u * 16 + lane)
            return tkx, tvx, _splat_lane(tkx, 8, tmp_v)

        def skip(_):
            return tk, tv, t9v

        return lax.cond(cnt[0] > 0, do_merge, skip, 0)

    tk0 = jnp.full((16,), _BIG, jnp.float32)
    tv0 = jnp.zeros((16,), jnp.int32)
    tk, tv, _ = lax.fori_loop(0, n_batches // 4, body,
                              (tk0, tv0, jnp.full((16,), _BIG, jnp.float32)))
    return tk, tv


def _sc_body(x0_h, y0_h, x1_h, y1_h, g0_h, g1_h, g2_h, g3_h,
             pmax_h, parg_h, recp_h, reci_h,
             cx_v, cy_v, st_v, g0_v, g1_v, g2_v, g3_v,
             ids_v, pov_v, bx0_v, by0_v, bx1_v, by1_v,
             recp_v, reci_v, best_v, bestg_v, tmp_v, sem):
    c = lax.axis_index("c")
    s = lax.axis_index("s")
    lane = _lane16()
    valid9 = lane < _K

    pltpu.sync_copy(x0_h, cx_v)
    pltpu.sync_copy(x1_h, st_v)

    def _cb(i, _):
        for u in range(4):
            o = i * 64 + u * 16
            cx_v[pl.ds(o, 16)] = (cx_v[pl.ds(o, 16)]
                                  + st_v[pl.ds(o, 16)]) * 0.5
        return 0

    lax.fori_loop(0, _N // 64, _cb, 0)
    pltpu.sync_copy(y0_h, cy_v)
    pltpu.sync_copy(y1_h, st_v)

    def _cb2(i, _):
        for u in range(4):
            o = i * 64 + u * 16
            cy_v[pl.ds(o, 16)] = (cy_v[pl.ds(o, 16)]
                                  + st_v[pl.ds(o, 16)]) * 0.5
        return 0

    lax.fori_loop(0, _N // 64, _cb2, 0)

    pltpu.sync_copy(g0_h, g0_v)
    pltpu.sync_copy(g1_h, g1_v)
    pltpu.sync_copy(g2_h, g2_v)
    pltpu.sync_copy(g3_h, g3_v)

    def _per_gt(k, _):
        g = c * _GPC + s + 16 * k
        gx0 = _splat_from(g0_v, g)
        gy0 = _splat_from(g1_v, g)
        gx1 = _splat_from(g2_v, g)
        gy1 = _splat_from(g3_v, g)
        gcx = (gx0 + gx1) * 0.5
        gcy = (gy0 + gy1) * 0.5

        for lvl in range(5):
            tk, tv = _scan_level(cx_v, cy_v, gcx, gcy, _STARTS[lvl],
                                 _LEVELS[lvl] // 16, tmp_v)
            ids_v[pl.ds(lvl * 16, 16)] = tv

        cps = [pltpu.async_copy(x0_h.at[ids_v], bx0_v, sem),
               pltpu.async_copy(y0_h.at[ids_v], by0_v, sem),
               pltpu.async_copy(x1_h.at[ids_v], bx1_v, sem),
               pltpu.async_copy(y1_h.at[ids_v], by1_v, sem)]
        for cp in cps:
            cp.wait()

        ovs = []
        ssum = jnp.zeros((16,), jnp.float32)
        for lvl in range(5):
            ax0 = bx0_v[pl.ds(lvl * 16, 16)]
            ay0 = by0_v[pl.ds(lvl * 16, 16)]
            ax1 = bx1_v[pl.ds(lvl * 16, 16)]
            ay1 = by1_v[pl.ds(lvl * 16, 16)]
            iw = jnp.maximum(jnp.minimum(ax1, gx1) - jnp.maximum(ax0, gx0),
                             0.0)
            ih = jnp.maximum(jnp.minimum(ay1, gy1) - jnp.maximum(ay0, gy0),
                             0.0)
            inter = iw * ih
            area_a = (ax1 - ax0) * (ay1 - ay0)
            area_g = (gx1 - gx0) * (gy1 - gy0)
            ov = inter / jnp.maximum(area_a + area_g - inter, 1e-6)
            acx = (ax0 + ax1) * 0.5
            acy = (ay0 + ay1) * 0.5
            dmin = jnp.minimum(jnp.minimum(acx - gx0, acy - gy0),
                               jnp.minimum(gx1 - acx, gy1 - acy))
            ovs.append((ov, dmin))
            ssum = ssum + _hsum(jnp.where(valid9, ov, 0.0), tmp_v)

        mean = ssum / 45.0
        vsum = jnp.zeros((16,), jnp.float32)
        for ov, _dm in ovs:
            dev = ov - mean
            vsum = vsum + _hsum(jnp.where(valid9, dev * dev, 0.0), tmp_v)
        thr = mean + _sqrt_newton(vsum / 44.0)

        for lvl in range(5):
            ov, dmin = ovs[lvl]
            pos = valid9 & (ov >= thr) & (dmin > 0.01)
            pov_v[pl.ds(lvl * 16, 16)] = jnp.where(pos, ov, -1.0)

        pltpu.sync_copy(pov_v, recp_h.at[pl.ds(g * _NREC, _NREC)])
        pltpu.sync_copy(ids_v, reci_h.at[pl.ds(g * _NREC, _NREC)])
        return 0

    n_gts = (_GPC - s + 15) // 16
    lax.fori_loop(0, n_gts, _per_gt, 0)

    plsc.subcore_barrier()
    pltpu.sync_copy(recp_h.at[pl.ds(c * _GPC * _NREC, _GPC * _NREC)], recp_v)
    pltpu.sync_copy(reci_h.at[pl.ds(c * _GPC * _NREC, _GPC * _NREC)], reci_v)
    base = s * _PER_TILE

    def _init(i, _):
        o = i * 16
        best_v[pl.ds(o, 16)] = jnp.full((16,), -_INF, jnp.float32)
        bestg_v[pl.ds(o, 16)] = jnp.zeros((16,), jnp.int32)
        return 0

    lax.fori_loop(0, _PER_TILE // 16, _init, 0)

    def _mb(t, _):
        off = t * 16
        pov = recp_v[pl.ds(off, 16)]
        ids = reci_v[pl.ds(off, 16)]
        loc = ids - base
        m = (loc >= 0) & (loc < _PER_TILE) & (pov >= 0.0)
        cur = plsc.load_gather(best_v, [loc], mask=m)
        upd = m & (pov > cur)
        gvec = jnp.full((16,), c * _GPC + t // 5, jnp.int32)
        plsc.store_scatter(best_v, [loc], pov, mask=upd)
        plsc.store_scatter(bestg_v, [loc], gvec, mask=upd)
        return 0

    lax.fori_loop(0, _GPC * 5, _mb, 0)

    pltpu.sync_copy(best_v, pmax_h.at[pl.ds(c * _NPC + base, _PER_TILE)])
    pltpu.sync_copy(bestg_v, parg_h.at[pl.ds(c * _NPC + base, _PER_TILE)])


def _combine_body(pm_ref, pa_ref, mo_ref, ag_ref):
    m0 = pm_ref[0:1, :]
    m1 = pm_ref[1:2, :]
    use0 = m0 >= m1
    mo_ref[...] = jnp.where(use0, m0, m1)
    ag_ref[...] = jnp.where(use0, pa_ref[0:1, :], pa_ref[1:2, :])


def kernel(anchors, gt_bboxes):
    at = anchors.T
    x0, y0, x1, y1 = at[0], at[1], at[2], at[3]
    gpad = jnp.zeros((4, _GPAD), jnp.float32).at[:, :_G].set(gt_bboxes.T)

    mesh = plsc.VectorSubcoreMesh(core_axis_name="c", subcore_axis_name="s",
                                  num_cores=2, num_subcores=16)
    sc = pl.kernel(
        _sc_body,
        out_type=[
            jax.ShapeDtypeStruct((2 * _NPC,), jnp.float32),
            jax.ShapeDtypeStruct((2 * _NPC,), jnp.int32),
            jax.ShapeDtypeStruct((_G * _NREC,), jnp.float32),
            jax.ShapeDtypeStruct((_G * _NREC,), jnp.int32),
        ],
        mesh=mesh,
        compiler_params=pltpu.CompilerParams(needs_layout_passes=False),
        scratch_types=[
            pltpu.VMEM((_N,), jnp.float32),
            pltpu.VMEM((_N,), jnp.float32),
            pltpu.VMEM((_N,), jnp.float32),
            pltpu.VMEM((_GPAD,), jnp.float32),
            pltpu.VMEM((_GPAD,), jnp.float32),
            pltpu.VMEM((_GPAD,), jnp.float32),
            pltpu.VMEM((_GPAD,), jnp.float32),
            pltpu.VMEM((_NREC,), jnp.int32),
            pltpu.VMEM((_NREC,), jnp.float32),
            pltpu.VMEM((_NREC,), jnp.float32),
            pltpu.VMEM((_NREC,), jnp.float32),
            pltpu.VMEM((_NREC,), jnp.float32),
            pltpu.VMEM((_NREC,), jnp.float32),
            pltpu.VMEM((_GPC * _NREC,), jnp.float32),
            pltpu.VMEM((_GPC * _NREC,), jnp.int32),
            pltpu.VMEM((_PER_TILE,), jnp.float32),
            pltpu.VMEM((_PER_TILE,), jnp.int32),
            pltpu.VMEM((16,), jnp.float32),
            pltpu.SemaphoreType.DMA,
        ],
    )
    pm, pa, _rp, _ri = sc(x0, y0, x1, y1,
                          gpad[0], gpad[1], gpad[2], gpad[3])

    mo, ag = pl.pallas_call(
        _combine_body,
        out_shape=(
            jax.ShapeDtypeStruct((1, _NPC), jnp.float32),
            jax.ShapeDtypeStruct((1, _NPC), jnp.int32),
        ),
    )(pm.reshape(2, _NPC), pa.reshape(2, _NPC))
    return mo[0, :_N], ag[0, :_N]


# speedup vs baseline: 2.0053x; 1.5113x over previous
import functools

import jax
import jax.numpy as jnp
from jax import lax
from jax.experimental import pallas as pl
from jax.experimental.pallas import tpu as pltpu
from jax.experimental.pallas import tpu_sc as plsc

_INF = 100000000.0
_LEVELS = (16384, 4096, 1024, 256, 64)
_STARTS = (0, 16384, 20480, 21504, 21760)
_N = 21824
_G = 100
_GPAD = 112
_K = 9
_NREC = 80
_PER_TILE = 1376
_NPC = 16 * _PER_TILE
_GPC = 50
_BIG = 3e38


def _lane16():
    return lax.broadcasted_iota(jnp.int32, (16,), 0)


def _splat_from(ref, j):
    return plsc.load_gather(ref, [jnp.full((16,), j, jnp.int32)])


def _splat_lane(vec, j, tmp_ref):
    tmp_ref[...] = vec
    return _splat_from(tmp_ref, j)


def _hsum(vec, tmp_ref):
    lane = _lane16()
    for m in (8, 4, 2, 1):
        tmp_ref[...] = vec
        vec = vec + plsc.load_gather(tmp_ref, [lane ^ m])
    return vec


def _sqrt_newton(x):
    i = lax.bitcast_convert_type(x, jnp.int32)
    y = lax.bitcast_convert_type((i >> 1) + jnp.int32(0x1FBD1DF6), jnp.float32)
    for _ in range(4):
        y = 0.5 * (y + x / y)
    return jnp.where(x > 0.0, y, 0.0)


def _merge_topk(tk, tv, d2, idx):
    dk, dv = plsc.sort_key_val(d2, idx)
    dk = lax.rev(dk, (0,))
    dv = lax.rev(dv, (0,))
    take_t = tk <= dk
    lk = jnp.where(take_t, tk, dk)
    lv = jnp.where(take_t, tv, dv)
    return plsc.sort_key_val(lk, lv)


def _scan_level(cx_v, cy_v, gcx, gcy, start, n_batches, tmp_v, unroll):
    lane = _lane16()

    def body(b, carry):
        tk, tv, t9v = carry
        off = start + b * (16 * unroll)
        d2s = []
        below = None
        for u in range(unroll):
            dx = cx_v[pl.ds(off + u * 16, 16)] - gcx
            dy = cy_v[pl.ds(off + u * 16, 16)] - gcy
            d2 = dx * dx + dy * dy
            d2s.append(d2)
            bl = d2 < t9v
            below = bl if below is None else (below | bl)
        cnt = plsc.all_reduce_population_count(below)

        def do_merge(_):
            tkx, tvx = tk, tv
            for u in range(unroll):
                tkx, tvx = _merge_topk(tkx, tvx, d2s[u], off + u * 16 + lane)
            return tkx, tvx, _splat_lane(tkx, 8, tmp_v)

        def skip(_):
            return tk, tv, t9v

        return lax.cond(cnt[0] > 0, do_merge, skip, 0)

    tk0 = jnp.full((16,), _BIG, jnp.float32)
    tv0 = jnp.zeros((16,), jnp.int32)
    tk, tv, _ = lax.fori_loop(0, n_batches // unroll, body,
                              (tk0, tv0, jnp.full((16,), _BIG, jnp.float32)))
    return tk, tv


def _sc_body(x0_h, y0_h, x1_h, y1_h, g0_h, g1_h, g2_h, g3_h,
             pmax_h, parg_h, recp_h, reci_h,
             cx_v, cy_v, st_v, g0_v, g1_v, g2_v, g3_v,
             ids_v, pov_v, bx0_v, by0_v, bx1_v, by1_v,
             recp_v, reci_v, best_v, bestg_v, tmp_v, sem):
    c = lax.axis_index("c")
    s = lax.axis_index("s")
    lane = _lane16()
    valid9 = lane < _K

    pltpu.sync_copy(x0_h, cx_v)
    pltpu.sync_copy(x1_h, st_v)

    def _cb(i, _):
        for u in range(4):
            o = i * 64 + u * 16
            cx_v[pl.ds(o, 16)] = (cx_v[pl.ds(o, 16)]
                                  + st_v[pl.ds(o, 16)]) * 0.5
        return 0

    lax.fori_loop(0, _N // 64, _cb, 0)
    pltpu.sync_copy(y0_h, cy_v)
    pltpu.sync_copy(y1_h, st_v)

    def _cb2(i, _):
        for u in range(4):
            o = i * 64 + u * 16
            cy_v[pl.ds(o, 16)] = (cy_v[pl.ds(o, 16)]
                                  + st_v[pl.ds(o, 16)]) * 0.5
        return 0

    lax.fori_loop(0, _N // 64, _cb2, 0)

    pltpu.sync_copy(g0_h, g0_v)
    pltpu.sync_copy(g1_h, g1_v)
    pltpu.sync_copy(g2_h, g2_v)
    pltpu.sync_copy(g3_h, g3_v)

    def _per_gt(k, _):
        g = c * _GPC + s + 16 * k
        gx0 = _splat_from(g0_v, g)
        gy0 = _splat_from(g1_v, g)
        gx1 = _splat_from(g2_v, g)
        gy1 = _splat_from(g3_v, g)
        gcx = (gx0 + gx1) * 0.5
        gcy = (gy0 + gy1) * 0.5

        for lvl in range(5):
            tk, tv = _scan_level(cx_v, cy_v, gcx, gcy, _STARTS[lvl],
                                 _LEVELS[lvl] // 16, tmp_v,
                                 8 if _LEVELS[lvl] >= 256 else 4)
            ids_v[pl.ds(lvl * 16, 16)] = tv

        cps = [pltpu.async_copy(x0_h.at[ids_v], bx0_v, sem),
               pltpu.async_copy(y0_h.at[ids_v], by0_v, sem),
               pltpu.async_copy(x1_h.at[ids_v], bx1_v, sem),
               pltpu.async_copy(y1_h.at[ids_v], by1_v, sem)]
        for cp in cps:
            cp.wait()

        ovs = []
        ssum = jnp.zeros((16,), jnp.float32)
        for lvl in range(5):
            ax0 = bx0_v[pl.ds(lvl * 16, 16)]
            ay0 = by0_v[pl.ds(lvl * 16, 16)]
            ax1 = bx1_v[pl.ds(lvl * 16, 16)]
            ay1 = by1_v[pl.ds(lvl * 16, 16)]
            iw = jnp.maximum(jnp.minimum(ax1, gx1) - jnp.maximum(ax0, gx0),
                             0.0)
            ih = jnp.maximum(jnp.minimum(ay1, gy1) - jnp.maximum(ay0, gy0),
                             0.0)
            inter = iw * ih
            area_a = (ax1 - ax0) * (ay1 - ay0)
            area_g = (gx1 - gx0) * (gy1 - gy0)
            ov = inter / jnp.maximum(area_a + area_g - inter, 1e-6)
            acx = (ax0 + ax1) * 0.5
            acy = (ay0 + ay1) * 0.5
            dmin = jnp.minimum(jnp.minimum(acx - gx0, acy - gy0),
                               jnp.minimum(gx1 - acx, gy1 - acy))
            ovs.append((ov, dmin))
            ssum = ssum + _hsum(jnp.where(valid9, ov, 0.0), tmp_v)

        mean = ssum / 45.0
        vsum = jnp.zeros((16,), jnp.float32)
        for ov, _dm in ovs:
            dev = ov - mean
            vsum = vsum + _hsum(jnp.where(valid9, dev * dev, 0.0), tmp_v)
        thr = mean + _sqrt_newton(vsum / 44.0)

        for lvl in range(5):
            ov, dmin = ovs[lvl]
            pos = valid9 & (ov >= thr) & (dmin > 0.01)
            pov_v[pl.ds(lvl * 16, 16)] = jnp.where(pos, ov, -1.0)

        pltpu.sync_copy(pov_v, recp_h.at[pl.ds(g * _NREC, _NREC)])
        pltpu.sync_copy(ids_v, reci_h.at[pl.ds(g * _NREC, _NREC)])
        return 0

    n_gts = (_GPC - s + 15) // 16
    lax.fori_loop(0, n_gts, _per_gt, 0)

    plsc.subcore_barrier()
    pltpu.sync_copy(recp_h.at[pl.ds(c * _GPC * _NREC, _GPC * _NREC)], recp_v)
    pltpu.sync_copy(reci_h.at[pl.ds(c * _GPC * _NREC, _GPC * _NREC)], reci_v)
    base = s * _PER_TILE

    def _init(i, _):
        o = i * 16
        best_v[pl.ds(o, 16)] = jnp.full((16,), -_INF, jnp.float32)
        bestg_v[pl.ds(o, 16)] = jnp.zeros((16,), jnp.int32)
        return 0

    lax.fori_loop(0, _PER_TILE // 16, _init, 0)

    def _mb(t, _):
        off = t * 16
        pov = recp_v[pl.ds(off, 16)]
        ids = reci_v[pl.ds(off, 16)]
        loc = ids - base
        m = (loc >= 0) & (loc < _PER_TILE) & (pov >= 0.0)
        cur = plsc.load_gather(best_v, [loc], mask=m)
        upd = m & (pov > cur)
        gvec = jnp.full((16,), c * _GPC + t // 5, jnp.int32)
        plsc.store_scatter(best_v, [loc], pov, mask=upd)
        plsc.store_scatter(bestg_v, [loc], gvec, mask=upd)
        return 0

    lax.fori_loop(0, _GPC * 5, _mb, 0)

    pltpu.sync_copy(best_v, pmax_h.at[pl.ds(c * _NPC + base, _PER_TILE)])
    pltpu.sync_copy(bestg_v, parg_h.at[pl.ds(c * _NPC + base, _PER_TILE)])


def _combine_body(pm_ref, pa_ref, mo_ref, ag_ref):
    m0 = pm_ref[0:1, :]
    m1 = pm_ref[1:2, :]
    use0 = m0 >= m1
    mo_ref[...] = jnp.where(use0, m0, m1)
    ag_ref[...] = jnp.where(use0, pa_ref[0:1, :], pa_ref[1:2, :])


def kernel(anchors, gt_bboxes):
    at = anchors.T
    x0, y0, x1, y1 = at[0], at[1], at[2], at[3]
    gpad = jnp.zeros((4, _GPAD), jnp.float32).at[:, :_G].set(gt_bboxes.T)

    mesh = plsc.VectorSubcoreMesh(core_axis_name="c", subcore_axis_name="s",
                                  num_cores=2, num_subcores=16)
    sc = pl.kernel(
        _sc_body,
        out_type=[
            jax.ShapeDtypeStruct((2 * _NPC,), jnp.float32),
            jax.ShapeDtypeStruct((2 * _NPC,), jnp.int32),
            jax.ShapeDtypeStruct((_G * _NREC,), jnp.float32),
            jax.ShapeDtypeStruct((_G * _NREC,), jnp.int32),
        ],
        mesh=mesh,
        compiler_params=pltpu.CompilerParams(needs_layout_passes=False),
        scratch_types=[
            pltpu.VMEM((_N,), jnp.float32),
            pltpu.VMEM((_N,), jnp.float32),
            pltpu.VMEM((_N,), jnp.float32),
            pltpu.VMEM((_GPAD,), jnp.float32),
            pltpu.VMEM((_GPAD,), jnp.float32),
            pltpu.VMEM((_GPAD,), jnp.float32),
            pltpu.VMEM((_GPAD,), jnp.float32),
            pltpu.VMEM((_NREC,), jnp.int32),
            pltpu.VMEM((_NREC,), jnp.float32),
            pltpu.VMEM((_NREC,), jnp.float32),
            pltpu.VMEM((_NREC,), jnp.float32),
            pltpu.VMEM((_NREC,), jnp.float32),
            pltpu.VMEM((_NREC,), jnp.float32),
            pltpu.VMEM((_GPC * _NREC,), jnp.float32),
            pltpu.VMEM((_GPC * _NREC,), jnp.int32),
            pltpu.VMEM((_PER_TILE,), jnp.float32),
            pltpu.VMEM((_PER_TILE,), jnp.int32),
            pltpu.VMEM((16,), jnp.float32),
            pltpu.SemaphoreType.DMA,
        ],
    )
    pm, pa, _rp, _ri = sc(x0, y0, x1, y1,
                          gpad[0], gpad[1], gpad[2], gpad[3])

    mo, ag = pl.pallas_call(
        _combine_body,
        out_shape=(
            jax.ShapeDtypeStruct((1, _NPC), jnp.float32),
            jax.ShapeDtypeStruct((1, _NPC), jnp.int32),
        ),
    )(pm.reshape(2, _NPC), pa.reshape(2, _NPC))
    return mo[0, :_N], ag[0, :_N]
